# Initial kernel scaffold; baseline (speedup 1.0000x reference)
import jax
import jax.numpy as jnp
from jax.experimental import pallas as pl


def kernel(x, edge_index, Ws0, Wn0, b0, Ws1, Wn1, b1, Ws2, Wn2, b2):
    raise NotImplementedError("write your pallas kernel here")


# speedup vs baseline: 4.8276x; 4.8276x over previous
import functools

import jax
import jax.numpy as jnp
from jax import lax
from jax.experimental import pallas as pl
from jax.experimental.pallas import tpu as pltpu
from jax.experimental.pallas import tpu_sc as plsc

N = 10000
E = 160000
NT = 16
NCORES = 2
CH = 80
EPT = E // NT
SCH = 2000
NSC = EPT // SCH
NKS = SCH // CH
FR = 624
TAIL = N - NT * FR
ZR = 48
NZ = FR // ZR


def _sc_agg(nch, dc):
    mesh = plsc.VectorSubcoreMesh(
        core_axis_name="c", subcore_axis_name="s",
        num_cores=NCORES, num_subcores=NT)
    out_type = jax.ShapeDtypeStruct((nch * N, dc), jnp.float32)
    scratch = [
        pltpu.VMEM((SCH,), jnp.int32),
        pltpu.VMEM((SCH,), jnp.int32),
        pltpu.VMEM((NKS, CH), jnp.int32),
        pltpu.VMEM((CH, dc), jnp.float32),
        pltpu.VMEM((ZR, dc), jnp.float32),
        pltpu.MemorySpace.VMEM_SHARED((N, dc), jnp.float32),
        pltpu.SemaphoreType.DMA,
    ]
    vpr = CH // 16

    def body(table, srcf, dst, out, sidx, dbuf, didx, rows, zbuf, acc, sem):
        cid = lax.axis_index("c")
        sid = lax.axis_index("s")
        z16 = jnp.zeros((16,), jnp.float32)

        def zb(i, _):
            r = i // (dc // 16)
            c = i % (dc // 16)
            zbuf[r, pl.ds(c * 16, 16)] = z16
            return 0
        lax.fori_loop(0, ZR * (dc // 16), zb, 0)

        for j in range(nch // NCORES):
            cch = NCORES * j + cid
            for z in range(NZ):
                pltpu.sync_copy(zbuf, acc.at[pl.ds(sid * FR + z * ZR, ZR)])

            @pl.when(sid == 0)
            def _():
                pltpu.sync_copy(zbuf.at[pl.ds(0, TAIL)],
                                acc.at[pl.ds(NT * FR, TAIL)])
            plsc.subcore_barrier()

            def sc_body(s, _):
                ebase = sid * EPT + s * SCH
                pltpu.sync_copy(srcf.at[pl.ds(cch * E + ebase, SCH)], sidx)
                pltpu.sync_copy(dst.at[pl.ds(ebase, SCH)], dbuf)

                def db(i, _):
                    r = i // vpr
                    c = (i % vpr) * 16
                    didx[r, pl.ds(c, 16)] = dbuf[pl.ds(i * 16, 16)]
                    return 0
                lax.fori_loop(0, SCH // 16, db, 0)

                def it_body(i, _):
                    pltpu.async_copy(
                        table.at[sidx.at[pl.ds(i * CH, CH)]], rows,
                        sem).wait()
                    pltpu.sync_copy(rows, acc.at[didx.at[i]], add=True)
                    return 0
                lax.fori_loop(0, NKS, it_body, 0)
                return 0
            lax.fori_loop(0, NSC, sc_body, 0)
            plsc.subcore_barrier()
            pltpu.sync_copy(
                acc.at[pl.ds(sid * FR, FR)],
                out.at[pl.ds(cch * N + sid * FR, FR)])

            @pl.when(sid == 0)
            def _():
                pltpu.sync_copy(acc.at[pl.ds(NT * FR, TAIL)],
                                out.at[pl.ds(cch * N + NT * FR, TAIL)])
            plsc.subcore_barrier()

    return pl.kernel(body, out_type=out_type, mesh=mesh,
                     scratch_types=scratch)


def _sc_deg():
    mesh = plsc.VectorSubcoreMesh(
        core_axis_name="c", subcore_axis_name="s",
        num_cores=NCORES, num_subcores=NT)
    out_type = jax.ShapeDtypeStruct((N, 128), jnp.float32)
    scratch = [
        pltpu.VMEM((SCH,), jnp.int32),
        pltpu.VMEM((NKS, CH), jnp.int32),
        pltpu.VMEM((CH, 128), jnp.float32),
        pltpu.VMEM((ZR, 128), jnp.float32),
        pltpu.MemorySpace.VMEM_SHARED((N, 128), jnp.float32),
    ]
    vpr = CH // 16

    def body(dst, out, dbuf, didx, ones, zbuf, acc):
        sid = lax.axis_index("s")
        z16 = jnp.zeros((16,), jnp.float32)
        o16 = jnp.ones((16,), jnp.float32)

        def fill(i, _):
            r = i // 8
            c = i % 8
            zbuf[r, pl.ds(c * 16, 16)] = z16
            return 0
        lax.fori_loop(0, ZR * 8, fill, 0)

        def fo(i, _):
            r = i // 8
            c = i % 8
            ones[r, pl.ds(c * 16, 16)] = o16
            return 0
        lax.fori_loop(0, CH * 8, fo, 0)

        for z in range(NZ):
            pltpu.sync_copy(zbuf, acc.at[pl.ds(sid * FR + z * ZR, ZR)])

        @pl.when(sid == 0)
        def _():
            pltpu.sync_copy(zbuf.at[pl.ds(0, TAIL)],
                            acc.at[pl.ds(NT * FR, TAIL)])
        plsc.subcore_barrier()

        def sc_body(s, _):
            ebase = sid * EPT + s * SCH
            pltpu.sync_copy(dst.at[pl.ds(ebase, SCH)], dbuf)

            def db(i, _):
                r = i // vpr
                c = (i % vpr) * 16
                didx[r, pl.ds(c, 16)] = dbuf[pl.ds(i * 16, 16)]
                return 0
            lax.fori_loop(0, SCH // 16, db, 0)

            def it_body(i, _):
                pltpu.sync_copy(ones, acc.at[didx.at[i]], add=True)
                return 0
            lax.fori_loop(0, NKS, it_body, 0)
            return 0
        lax.fori_loop(0, NSC, sc_body, 0)
        plsc.subcore_barrier()
        pltpu.sync_copy(acc.at[pl.ds(sid * FR, FR)],
                        out.at[pl.ds(sid * FR, FR)])

        @pl.when(sid == 0)
        def _():
            pltpu.sync_copy(acc.at[pl.ds(NT * FR, TAIL)],
                            out.at[pl.ds(NT * FR, TAIL)])

    return pl.kernel(body, out_type=out_type, mesh=mesh,
                     scratch_types=scratch)


ES_EPT0 = 5040


def _sc_agg_esplit(dc):
    mesh = plsc.VectorSubcoreMesh(
        core_axis_name="c", subcore_axis_name="s",
        num_cores=NCORES, num_subcores=NT)
    out_type = jax.ShapeDtypeStruct((NCORES * N, dc), jnp.float32)
    scratch = [
        pltpu.VMEM((ES_EPT0,), jnp.int32),
        pltpu.VMEM((ES_EPT0,), jnp.int32),
        pltpu.VMEM((ES_EPT0 // CH, CH), jnp.int32),
        pltpu.VMEM((CH, dc), jnp.float32),
        pltpu.VMEM((ZR, dc), jnp.float32),
        pltpu.MemorySpace.VMEM_SHARED((N, dc), jnp.float32),
        pltpu.SemaphoreType.DMA,
    ]
    vpr = CH // 16

    def body(table, src, dst, out, sidx, dbuf, didx, rows, zbuf, acc, sem):
        cid = lax.axis_index("c")
        sid = lax.axis_index("s")
        z16 = jnp.zeros((16,), jnp.float32)

        def zb(i, _):
            r = i // (dc // 16)
            c = i % (dc // 16)
            zbuf[r, pl.ds(c * 16, 16)] = z16
            return 0
        lax.fori_loop(0, ZR * (dc // 16), zb, 0)

        epc = ES_EPT0 - CH * cid
        base = cid * (NT * ES_EPT0) + sid * epc
        pltpu.sync_copy(src.at[pl.ds(base, ES_EPT0 - CH)],
                        sidx.at[pl.ds(0, ES_EPT0 - CH)])
        pltpu.sync_copy(dst.at[pl.ds(base, ES_EPT0 - CH)],
                        dbuf.at[pl.ds(0, ES_EPT0 - CH)])

        @pl.when(cid == 0)
        def _():
            pltpu.sync_copy(src.at[pl.ds(base + ES_EPT0 - CH, CH)],
                            sidx.at[pl.ds(ES_EPT0 - CH, CH)])
            pltpu.sync_copy(dst.at[pl.ds(base + ES_EPT0 - CH, CH)],
                            dbuf.at[pl.ds(ES_EPT0 - CH, CH)])

        def db(i, _):
            r = i // vpr
            c = (i % vpr) * 16
            didx[r, pl.ds(c, 16)] = dbuf[pl.ds(i * 16, 16)]
            return 0
        lax.fori_loop(0, ES_EPT0 // 16, db, 0)

        for z in range(NZ):
            pltpu.sync_copy(zbuf, acc.at[pl.ds(sid * FR + z * ZR, ZR)])

        @pl.when(sid == 0)
        def _():
            pltpu.sync_copy(zbuf.at[pl.ds(0, TAIL)],
                            acc.at[pl.ds(NT * FR, TAIL)])
        plsc.subcore_barrier()

        def it_body(i, _):
            pltpu.async_copy(
                table.at[sidx.at[pl.ds(i * CH, CH)]], rows, sem).wait()
            pltpu.sync_copy(rows, acc.at[didx.at[i]], add=True)
            return 0
        lax.fori_loop(0, epc // CH, it_body, 0)
        plsc.subcore_barrier()
        pltpu.sync_copy(acc.at[pl.ds(sid * FR, FR)],
                        out.at[pl.ds(cid * N + sid * FR, FR)])

        @pl.when(sid == 0)
        def _():
            pltpu.sync_copy(acc.at[pl.ds(NT * FR, TAIL)],
                            out.at[pl.ds(cid * N + NT * FR, TAIL)])

    return pl.kernel(body, out_type=out_type, mesh=mesh,
                     scratch_types=scratch)


def _tc_layer(nchi, dc, do, ncho, dco, relu, bn=400):
    di = nchi * dc

    def body(h_ref, s_ref, deg_ref, ws_ref, wn_ref, b_ref, out_ref):
        acc = jnp.zeros((bn, do), jnp.float32)
        accn = jnp.zeros((bn, do), jnp.float32)
        for c in range(nchi):
            acc = acc + jnp.dot(h_ref[c], ws_ref[pl.ds(c * dc, dc), :],
                                preferred_element_type=jnp.float32)
            accn = accn + jnp.dot(s_ref[c], wn_ref[pl.ds(c * dc, dc), :],
                                  preferred_element_type=jnp.float32)
        r = 1.0 / jnp.maximum(deg_ref[...], 1.0)
        res = acc + accn * r + b_ref[...]
        if relu:
            res = jnp.maximum(res, 0.0)
        for c2 in range(ncho):
            out_ref[c2] = res[:, c2 * dco:(c2 + 1) * dco]

    grid = (N // bn,)
    return pl.pallas_call(
        body,
        grid=grid,
        in_specs=[
            pl.BlockSpec((nchi, bn, dc), lambda i: (0, i, 0)),
            pl.BlockSpec((nchi, bn, dc), lambda i: (0, i, 0)),
            pl.BlockSpec((bn, 1), lambda i: (i, 0)),
            pl.BlockSpec((di, do), lambda i: (0, 0)),
            pl.BlockSpec((di, do), lambda i: (0, 0)),
            pl.BlockSpec((1, do), lambda i: (0, 0)),
        ],
        out_specs=pl.BlockSpec((ncho, bn, dco), lambda i: (0, i, 0)),
        out_shape=jax.ShapeDtypeStruct((ncho, N, dco), jnp.float32),
    )


def _tc_transform(nchi, dc, do, ncho, dco, bn=400):
    def body(h_ref, w_ref, out_ref):
        acc = jnp.zeros((bn, do), jnp.float32)
        for c in range(nchi):
            acc = acc + jnp.dot(h_ref[c], w_ref[pl.ds(c * dc, dc), :],
                                preferred_element_type=jnp.float32)
        for c2 in range(ncho):
            out_ref[c2] = acc[:, c2 * dco:(c2 + 1) * dco]

    return pl.pallas_call(
        body,
        grid=(N // bn,),
        in_specs=[
            pl.BlockSpec((nchi, bn, dc), lambda i: (0, i, 0)),
            pl.BlockSpec((nchi * dc, do), lambda i: (0, 0)),
        ],
        out_specs=pl.BlockSpec((ncho, bn, dco), lambda i: (0, i, 0)),
        out_shape=jax.ShapeDtypeStruct((ncho, N, dco), jnp.float32),
    )


def _tc_final(nchi, dc, do, bn=400):
    def body(h_ref, s_ref, deg_ref, ws_ref, b_ref, out_ref):
        acc = jnp.zeros((bn, do), jnp.float32)
        for c in range(nchi):
            acc = acc + jnp.dot(h_ref[c], ws_ref[pl.ds(c * dc, dc), :],
                                preferred_element_type=jnp.float32)
        mean = s_ref[0] + s_ref[1]
        r = 1.0 / jnp.maximum(deg_ref[...], 1.0)
        out_ref[...] = acc + mean * r + b_ref[...]

    return pl.pallas_call(
        body,
        grid=(N // bn,),
        in_specs=[
            pl.BlockSpec((nchi, bn, dc), lambda i: (0, i, 0)),
            pl.BlockSpec((2, bn, do), lambda i: (0, i, 0)),
            pl.BlockSpec((bn, 1), lambda i: (i, 0)),
            pl.BlockSpec((nchi * dc, do), lambda i: (0, 0)),
            pl.BlockSpec((1, do), lambda i: (0, 0)),
        ],
        out_specs=pl.BlockSpec((bn, do), lambda i: (i, 0)),
        out_shape=jax.ShapeDtypeStruct((N, do), jnp.float32),
    )


def kernel(x, edge_index, Ws0, Wn0, b0, Ws1, Wn1, b1, Ws2, Wn2, b2):
    src = edge_index[0]
    dst = edge_index[1]

    def srcf(nch):
        off = (jnp.arange(nch, dtype=jnp.int32) * N)[:, None]
        return (src[None, :] + off).reshape(nch * E)

    srcf2 = srcf(2)
    srcf4 = srcf(4)

    x_s = x.reshape(N, 2, 128).transpose(1, 0, 2)

    deg = _sc_deg()(dst)[:, :1]
    S0 = _sc_agg(2, 128)(x_s.reshape(2 * N, 128), srcf2, dst)

    h1_s = _tc_layer(2, 128, 512, 4, 128, True)(
        x_s, S0.reshape(2, N, 128), deg, Ws0, Wn0, b0.reshape(1, -1))

    S1 = _sc_agg(4, 128)(h1_s.reshape(4 * N, 128), srcf4, dst)

    h2_s = _tc_layer(4, 128, 512, 4, 128, True)(
        h1_s, S1.reshape(4, N, 128), deg, Ws1, Wn1, b1.reshape(1, -1))

    z2 = _tc_transform(4, 128, 128, 1, 128)(h2_s, Wn2)

    S2 = _sc_agg_esplit(128)(z2.reshape(N, 128), src, dst)

    out = _tc_final(4, 128, 128)(
        h2_s, S2.reshape(2, N, 128), deg, Ws2, b2.reshape(1, -1))
    return out


# speedup vs baseline: 7.0379x; 1.4578x over previous
import functools

import jax
import jax.numpy as jnp
from jax import lax
from jax.experimental import pallas as pl
from jax.experimental.pallas import tpu as pltpu
from jax.experimental.pallas import tpu_sc as plsc

N = 10000
E = 160000
NT = 16
NCORES = 2
CH = 80
EPT = E // NT
SCH = 2000
NSC = EPT // SCH
NKS = SCH // CH
FR = 624
TAIL = N - NT * FR
ZR = 48
NZ = FR // ZR


def _sc_agg(nch, dc):
    mesh = plsc.VectorSubcoreMesh(
        core_axis_name="c", subcore_axis_name="s",
        num_cores=NCORES, num_subcores=NT)
    out_type = jax.ShapeDtypeStruct((nch * N, dc), jnp.float32)
    scratch = [
        pltpu.VMEM((SCH,), jnp.int32),
        pltpu.VMEM((SCH,), jnp.int32),
        pltpu.VMEM((NKS, CH), jnp.int32),
        pltpu.VMEM((CH, dc), jnp.float32),
        pltpu.VMEM((CH, dc), jnp.float32),
        pltpu.VMEM((ZR, dc), jnp.float32),
        pltpu.MemorySpace.VMEM_SHARED((N, dc), jnp.float32),
        pltpu.SemaphoreType.DMA,
        pltpu.SemaphoreType.DMA,
        pltpu.SemaphoreType.DMA,
    ]
    vpr = CH // 16

    def body(table, srcf, dst, out, sidx, dbuf, didx, rows0, rows1, zbuf,
             acc, gsem0, gsem1, ssem):
        cid = lax.axis_index("c")
        sid = lax.axis_index("s")
        z16 = jnp.zeros((16,), jnp.float32)

        def zb(i, _):
            r = i // (dc // 16)
            c = i % (dc // 16)
            zbuf[r, pl.ds(c * 16, 16)] = z16
            return 0
        lax.fori_loop(0, ZR * (dc // 16), zb, 0)

        for j in range(nch // NCORES):
            cch = NCORES * j + cid
            for z in range(NZ):
                pltpu.sync_copy(zbuf, acc.at[pl.ds(sid * FR + z * ZR, ZR)])

            @pl.when(sid == 0)
            def _():
                pltpu.sync_copy(zbuf.at[pl.ds(0, TAIL)],
                                acc.at[pl.ds(NT * FR, TAIL)])
            plsc.subcore_barrier()

            def sc_body(s, _):
                ebase = sid * EPT + s * SCH
                pltpu.sync_copy(srcf.at[pl.ds(cch * E + ebase, SCH)], sidx)
                pltpu.async_copy(
                    table.at[sidx.at[pl.ds(0, CH)]], rows0, gsem0)
                pltpu.sync_copy(dst.at[pl.ds(ebase, SCH)], dbuf)

                def db(i, _):
                    r = i // vpr
                    c = (i % vpr) * 16
                    didx[r, pl.ds(c, 16)] = dbuf[pl.ds(i * 16, 16)]
                    return 0
                lax.fori_loop(0, SCH // 16, db, 0)

                def pair(p, _):
                    i0 = 2 * p
                    pltpu.make_async_copy(
                        table.at[sidx.at[pl.ds(0, CH)]], rows0, gsem0).wait()
                    pltpu.async_copy(
                        table.at[sidx.at[pl.ds((i0 + 1) * CH, CH)]],
                        rows1, gsem1)
                    pltpu.async_copy(
                        rows0, acc.at[didx.at[i0]], ssem, add=True).wait()
                    pltpu.async_copy(
                        table.at[sidx.at[pl.ds((i0 + 2) * CH, CH)]],
                        rows0, gsem0)
                    pltpu.make_async_copy(
                        table.at[sidx.at[pl.ds(0, CH)]], rows1, gsem1).wait()
                    pltpu.async_copy(
                        rows1, acc.at[didx.at[i0 + 1]], ssem, add=True).wait()
                    return 0
                lax.fori_loop(0, NKS // 2, pair, 0)
                pltpu.make_async_copy(
                    table.at[sidx.at[pl.ds(0, CH)]], rows0, gsem0).wait()
                pltpu.async_copy(
                    rows0, acc.at[didx.at[NKS - 1]], ssem, add=True).wait()
                return 0
            lax.fori_loop(0, NSC, sc_body, 0)
            plsc.subcore_barrier()
            pltpu.sync_copy(
                acc.at[pl.ds(sid * FR, FR)],
                out.at[pl.ds(cch * N + sid * FR, FR)])

            @pl.when(sid == 0)
            def _():
                pltpu.sync_copy(acc.at[pl.ds(NT * FR, TAIL)],
                                out.at[pl.ds(cch * N + NT * FR, TAIL)])
            plsc.subcore_barrier()

    return pl.kernel(body, out_type=out_type, mesh=mesh,
                     scratch_types=scratch)


def _sc_deg():
    mesh = plsc.VectorSubcoreMesh(
        core_axis_name="c", subcore_axis_name="s",
        num_cores=NCORES, num_subcores=NT)
    out_type = jax.ShapeDtypeStruct((N, 128), jnp.float32)
    scratch = [
        pltpu.VMEM((EPT,), jnp.int32),
        pltpu.VMEM((EPT // CH, CH), jnp.int32),
        pltpu.VMEM((CH, 128), jnp.float32),
        pltpu.VMEM((ZR, 128), jnp.float32),
        pltpu.MemorySpace.VMEM_SHARED((N, 128), jnp.float32),
        pltpu.SemaphoreType.DMA,
    ]
    vpr = CH // 16

    def body(dst, out, dbuf, didx, ones, zbuf, acc, ssem):
        sid = lax.axis_index("s")
        z16 = jnp.zeros((16,), jnp.float32)
        o16 = jnp.ones((16,), jnp.float32)

        def fill(i, _):
            r = i // 8
            c = i % 8
            zbuf[r, pl.ds(c * 16, 16)] = z16
            return 0
        lax.fori_loop(0, ZR * 8, fill, 0)

        def fo(i, _):
            r = i // 8
            c = i % 8
            ones[r, pl.ds(c * 16, 16)] = o16
            return 0
        lax.fori_loop(0, CH * 8, fo, 0)

        for z in range(NZ):
            pltpu.sync_copy(zbuf, acc.at[pl.ds(sid * FR + z * ZR, ZR)])

        @pl.when(sid == 0)
        def _():
            pltpu.sync_copy(zbuf.at[pl.ds(0, TAIL)],
                            acc.at[pl.ds(NT * FR, TAIL)])
        plsc.subcore_barrier()

        pltpu.sync_copy(dst.at[pl.ds(sid * EPT, EPT)], dbuf)

        def db(i, _):
            r = i // vpr
            c = (i % vpr) * 16
            didx[r, pl.ds(c, 16)] = dbuf[pl.ds(i * 16, 16)]
            return 0
        lax.fori_loop(0, EPT // 16, db, 0)

        def grp(g, _):
            def fire(i, _):
                pltpu.async_copy(ones, acc.at[didx.at[g * NKS + i]],
                                 ssem, add=True)
                return 0
            lax.fori_loop(0, NKS, fire, 0)

            def drain(i, _):
                pltpu.make_async_copy(ones, acc.at[didx.at[0]], ssem).wait()
                return 0
            lax.fori_loop(0, NKS, drain, 0)
            return 0
        lax.fori_loop(0, EPT // CH // NKS, grp, 0)
        plsc.subcore_barrier()
        pltpu.sync_copy(acc.at[pl.ds(sid * FR, FR)],
                        out.at[pl.ds(sid * FR, FR)])

        @pl.when(sid == 0)
        def _():
            pltpu.sync_copy(acc.at[pl.ds(NT * FR, TAIL)],
                            out.at[pl.ds(NT * FR, TAIL)])

    return pl.kernel(body, out_type=out_type, mesh=mesh,
                     scratch_types=scratch)


ES_EPT0 = 5040


def _sc_agg_esplit(dc):
    mesh = plsc.VectorSubcoreMesh(
        core_axis_name="c", subcore_axis_name="s",
        num_cores=NCORES, num_subcores=NT)
    out_type = jax.ShapeDtypeStruct((NCORES * N, dc), jnp.float32)
    scratch = [
        pltpu.VMEM((ES_EPT0,), jnp.int32),
        pltpu.VMEM((ES_EPT0,), jnp.int32),
        pltpu.VMEM((ES_EPT0 // CH, CH), jnp.int32),
        pltpu.VMEM((CH, dc), jnp.float32),
        pltpu.VMEM((CH, dc), jnp.float32),
        pltpu.VMEM((ZR, dc), jnp.float32),
        pltpu.MemorySpace.VMEM_SHARED((N, dc), jnp.float32),
        pltpu.SemaphoreType.DMA,
        pltpu.SemaphoreType.DMA,
        pltpu.SemaphoreType.DMA,
    ]
    vpr = CH // 16

    def body(table, src, dst, out, sidx, dbuf, didx, rows0, rows1, zbuf,
             acc, gsem0, gsem1, ssem):
        cid = lax.axis_index("c")
        sid = lax.axis_index("s")
        z16 = jnp.zeros((16,), jnp.float32)

        def zb(i, _):
            r = i // (dc // 16)
            c = i % (dc // 16)
            zbuf[r, pl.ds(c * 16, 16)] = z16
            return 0
        lax.fori_loop(0, ZR * (dc // 16), zb, 0)

        epc = ES_EPT0 - CH * cid
        base = cid * (NT * ES_EPT0) + sid * epc
        pltpu.sync_copy(src.at[pl.ds(base, ES_EPT0 - CH)],
                        sidx.at[pl.ds(0, ES_EPT0 - CH)])
        pltpu.sync_copy(dst.at[pl.ds(base, ES_EPT0 - CH)],
                        dbuf.at[pl.ds(0, ES_EPT0 - CH)])

        @pl.when(cid == 0)
        def _():
            pltpu.sync_copy(src.at[pl.ds(base + ES_EPT0 - CH, CH)],
                            sidx.at[pl.ds(ES_EPT0 - CH, CH)])
            pltpu.sync_copy(dst.at[pl.ds(base + ES_EPT0 - CH, CH)],
                            dbuf.at[pl.ds(ES_EPT0 - CH, CH)])

        def db(i, _):
            r = i // vpr
            c = (i % vpr) * 16
            didx[r, pl.ds(c, 16)] = dbuf[pl.ds(i * 16, 16)]
            return 0
        lax.fori_loop(0, ES_EPT0 // 16, db, 0)

        for z in range(NZ):
            pltpu.sync_copy(zbuf, acc.at[pl.ds(sid * FR + z * ZR, ZR)])

        @pl.when(sid == 0)
        def _():
            pltpu.sync_copy(zbuf.at[pl.ds(0, TAIL)],
                            acc.at[pl.ds(NT * FR, TAIL)])
        plsc.subcore_barrier()

        nst = epc // CH
        pltpu.async_copy(table.at[sidx.at[pl.ds(0, CH)]], rows0, gsem0)

        def pair(p, _):
            i0 = 2 * p
            pltpu.make_async_copy(
                table.at[sidx.at[pl.ds(0, CH)]], rows0, gsem0).wait()
            pltpu.async_copy(
                table.at[sidx.at[pl.ds((i0 + 1) * CH, CH)]], rows1, gsem1)
            pltpu.async_copy(
                rows0, acc.at[didx.at[i0]], ssem, add=True).wait()

            @pl.when(i0 + 2 < nst)
            def _():
                pltpu.async_copy(
                    table.at[sidx.at[pl.ds((i0 + 2) * CH, CH)]], rows0,
                    gsem0)
            pltpu.make_async_copy(
                table.at[sidx.at[pl.ds(0, CH)]], rows1, gsem1).wait()
            pltpu.async_copy(
                rows1, acc.at[didx.at[i0 + 1]], ssem, add=True).wait()
            return 0
        lax.fori_loop(0, (ES_EPT0 // CH - 1) // 2, pair, 0)

        @pl.when(cid == 0)
        def _():
            pltpu.make_async_copy(
                table.at[sidx.at[pl.ds(0, CH)]], rows0, gsem0).wait()
            pltpu.async_copy(
                rows0, acc.at[didx.at[ES_EPT0 // CH - 1]], ssem,
                add=True).wait()
        plsc.subcore_barrier()
        pltpu.sync_copy(acc.at[pl.ds(sid * FR, FR)],
                        out.at[pl.ds(cid * N + sid * FR, FR)])

        @pl.when(sid == 0)
        def _():
            pltpu.sync_copy(acc.at[pl.ds(NT * FR, TAIL)],
                            out.at[pl.ds(cid * N + NT * FR, TAIL)])

    return pl.kernel(body, out_type=out_type, mesh=mesh,
                     scratch_types=scratch)


def _tc_layer(nchi, dc, do, ncho, dco, relu, bn=400):
    di = nchi * dc

    def body(h_ref, s_ref, deg_ref, ws_ref, wn_ref, b_ref, out_ref):
        acc = jnp.zeros((bn, do), jnp.float32)
        accn = jnp.zeros((bn, do), jnp.float32)
        for c in range(nchi):
            acc = acc + jnp.dot(h_ref[c], ws_ref[pl.ds(c * dc, dc), :],
                                preferred_element_type=jnp.float32)
            accn = accn + jnp.dot(s_ref[c], wn_ref[pl.ds(c * dc, dc), :],
                                  preferred_element_type=jnp.float32)
        r = 1.0 / jnp.maximum(deg_ref[...], 1.0)
        res = acc + accn * r + b_ref[...]
        if relu:
            res = jnp.maximum(res, 0.0)
        for c2 in range(ncho):
            out_ref[c2] = res[:, c2 * dco:(c2 + 1) * dco]

    grid = (N // bn,)
    return pl.pallas_call(
        body,
        grid=grid,
        in_specs=[
            pl.BlockSpec((nchi, bn, dc), lambda i: (0, i, 0)),
            pl.BlockSpec((nchi, bn, dc), lambda i: (0, i, 0)),
            pl.BlockSpec((bn, 1), lambda i: (i, 0)),
            pl.BlockSpec((di, do), lambda i: (0, 0)),
            pl.BlockSpec((di, do), lambda i: (0, 0)),
            pl.BlockSpec((1, do), lambda i: (0, 0)),
        ],
        out_specs=pl.BlockSpec((ncho, bn, dco), lambda i: (0, i, 0)),
        out_shape=jax.ShapeDtypeStruct((ncho, N, dco), jnp.float32),
    )


def _tc_transform(nchi, dc, do, ncho, dco, bn=400):
    def body(h_ref, w_ref, out_ref):
        acc = jnp.zeros((bn, do), jnp.float32)
        for c in range(nchi):
            acc = acc + jnp.dot(h_ref[c], w_ref[pl.ds(c * dc, dc), :],
                                preferred_element_type=jnp.float32)
        for c2 in range(ncho):
            out_ref[c2] = acc[:, c2 * dco:(c2 + 1) * dco]

    return pl.pallas_call(
        body,
        grid=(N // bn,),
        in_specs=[
            pl.BlockSpec((nchi, bn, dc), lambda i: (0, i, 0)),
            pl.BlockSpec((nchi * dc, do), lambda i: (0, 0)),
        ],
        out_specs=pl.BlockSpec((ncho, bn, dco), lambda i: (0, i, 0)),
        out_shape=jax.ShapeDtypeStruct((ncho, N, dco), jnp.float32),
    )


def _tc_final(nchi, dc, do, bn=400):
    def body(h_ref, s_ref, deg_ref, ws_ref, b_ref, out_ref):
        acc = jnp.zeros((bn, do), jnp.float32)
        for c in range(nchi):
            acc = acc + jnp.dot(h_ref[c], ws_ref[pl.ds(c * dc, dc), :],
                                preferred_element_type=jnp.float32)
        mean = s_ref[0] + s_ref[1]
        r = 1.0 / jnp.maximum(deg_ref[...], 1.0)
        out_ref[...] = acc + mean * r + b_ref[...]

    return pl.pallas_call(
        body,
        grid=(N // bn,),
        in_specs=[
            pl.BlockSpec((nchi, bn, dc), lambda i: (0, i, 0)),
            pl.BlockSpec((2, bn, do), lambda i: (0, i, 0)),
            pl.BlockSpec((bn, 1), lambda i: (i, 0)),
            pl.BlockSpec((nchi * dc, do), lambda i: (0, 0)),
            pl.BlockSpec((1, do), lambda i: (0, 0)),
        ],
        out_specs=pl.BlockSpec((bn, do), lambda i: (i, 0)),
        out_shape=jax.ShapeDtypeStruct((N, do), jnp.float32),
    )


def kernel(x, edge_index, Ws0, Wn0, b0, Ws1, Wn1, b1, Ws2, Wn2, b2):
    src = edge_index[0]
    dst = edge_index[1]

    def srcf(nch):
        off = (jnp.arange(nch, dtype=jnp.int32) * N)[:, None]
        return (src[None, :] + off).reshape(nch * E)

    srcf2 = srcf(2)
    srcf4 = srcf(4)

    x_s = x.reshape(N, 2, 128).transpose(1, 0, 2)

    deg = _sc_deg()(dst)[:, :1]
    S0 = _sc_agg(2, 128)(x_s.reshape(2 * N, 128), srcf2, dst)

    h1_s = _tc_layer(2, 128, 512, 4, 128, True)(
        x_s, S0.reshape(2, N, 128), deg, Ws0, Wn0, b0.reshape(1, -1))

    S1 = _sc_agg(4, 128)(h1_s.reshape(4 * N, 128), srcf4, dst)

    h2_s = _tc_layer(4, 128, 512, 4, 128, True)(
        h1_s, S1.reshape(4, N, 128), deg, Ws1, Wn1, b1.reshape(1, -1))

    z2 = _tc_transform(4, 128, 128, 1, 128)(h2_s, Wn2)

    S2 = _sc_agg_esplit(128)(z2.reshape(N, 128), src, dst)

    out = _tc_final(4, 128, 128)(
        h2_s, S2.reshape(2, N, 128), deg, Ws2, b2.reshape(1, -1))
    return out


# speedup vs baseline: 7.9247x; 1.1260x over previous
import functools

import jax
import jax.numpy as jnp
from jax import lax
from jax.experimental import pallas as pl
from jax.experimental.pallas import tpu as pltpu
from jax.experimental.pallas import tpu_sc as plsc

N = 10000
E = 160000
NT = 16
NCORES = 2
CH = 80
EPT = E // NT
SCH = 2000
NSC = EPT // SCH
NKS = SCH // CH
FR = 624
TAIL = N - NT * FR
ZR = 80
NZ = FR // ZR
ZREM = FR - NZ * ZR


def _sc_agg(nch, dc):
    mesh = plsc.VectorSubcoreMesh(
        core_axis_name="c", subcore_axis_name="s",
        num_cores=NCORES, num_subcores=NT)
    out_type = jax.ShapeDtypeStruct((nch * N, dc), jnp.float32)
    scratch = [
        pltpu.VMEM((SCH,), jnp.int32),
        pltpu.VMEM((SCH,), jnp.int32),
        pltpu.VMEM((NKS, CH), jnp.int32),
        [pltpu.VMEM((CH, dc), jnp.float32) for _ in range(3)],
        pltpu.VMEM((ZR, dc), jnp.float32),
        pltpu.MemorySpace.VMEM_SHARED((N, dc), jnp.float32),
        [pltpu.SemaphoreType.DMA for _ in range(3)],
        [pltpu.SemaphoreType.DMA for _ in range(3)],
    ]
    vpr = CH // 16
    SB = CH * dc * 4

    def body(table, srcf, dst, out, sidx, dbuf, didx, rows, zbuf,
             acc, gsem, ssem):
        cid = lax.axis_index("c")
        sid = lax.axis_index("s")
        z16 = jnp.zeros((16,), jnp.float32)

        def zb(i, _):
            r = i // (dc // 16)
            c = i % (dc // 16)
            zbuf[r, pl.ds(c * 16, 16)] = z16
            return 0
        lax.fori_loop(0, ZR * (dc // 16), zb, 0)

        for j in range(nch // NCORES):
            cch = NCORES * j + cid
            for z in range(NZ):
                pltpu.sync_copy(zbuf, acc.at[pl.ds(sid * FR + z * ZR, ZR)])
            pltpu.sync_copy(zbuf.at[pl.ds(0, ZREM)],
                            acc.at[pl.ds(sid * FR + NZ * ZR, ZREM)])

            @pl.when(sid == 0)
            def _():
                pltpu.sync_copy(zbuf.at[pl.ds(0, TAIL)],
                                acc.at[pl.ds(NT * FR, TAIL)])
            plsc.subcore_barrier()

            def sc_body(s, _):
                ebase = sid * EPT + s * SCH
                pltpu.sync_copy(srcf.at[pl.ds(cch * E + ebase, SCH)], sidx)
                pltpu.async_copy(
                    table.at[sidx.at[pl.ds(0, CH)]], rows[0], gsem[0])
                pltpu.sync_copy(dst.at[pl.ds(ebase, SCH)], dbuf)

                def db(i, _):
                    r = i // vpr
                    c = (i % vpr) * 16
                    didx[r, pl.ds(c, 16)] = dbuf[pl.ds(i * 16, 16)]
                    return 0
                lax.fori_loop(0, SCH // 16, db, 0)

                pltpu.async_copy(
                    table.at[sidx.at[pl.ds(CH, CH)]], rows[1], gsem[1])
                pltpu.async_copy(zbuf, acc.at[didx.at[0]], ssem[2],
                                 add=True)

                def gwait(b):
                    pltpu.make_async_copy(
                        table.at[sidx.at[pl.ds(0, CH)]], rows[b],
                        gsem[b]).wait()

                def swait(b):
                    pltpu.make_async_copy(
                        rows[b], acc.at[didx.at[0]], ssem[b]).wait()

                def tri(q, _):
                    for k in range(3):
                        i = 3 * q + k
                        bp = (k - 1) % 3
                        gwait(k)
                        pltpu.async_copy(rows[k], acc.at[didx.at[i]],
                                         ssem[k], add=True)
                        swait(bp)

                        @pl.when(i + 2 < NKS)
                        def _():
                            pltpu.async_copy(
                                table.at[sidx.at[pl.ds((i + 2) * CH, CH)]],
                                rows[bp], gsem[bp])
                    return 0
                lax.fori_loop(0, NKS // 3, tri, 0)
                gwait(0)
                pltpu.async_copy(rows[0], acc.at[didx.at[NKS - 1]],
                                 ssem[0], add=True)
                swait(2)
                swait(0)
                return 0
            lax.fori_loop(0, NSC, sc_body, 0)
            plsc.subcore_barrier()
            pltpu.sync_copy(
                acc.at[pl.ds(sid * FR, FR)],
                out.at[pl.ds(cch * N + sid * FR, FR)])

            @pl.when(sid == 0)
            def _():
                pltpu.sync_copy(acc.at[pl.ds(NT * FR, TAIL)],
                                out.at[pl.ds(cch * N + NT * FR, TAIL)])
            plsc.subcore_barrier()

    return pl.kernel(body, out_type=out_type, mesh=mesh,
                     scratch_types=scratch)


def _sc_deg():
    mesh = plsc.VectorSubcoreMesh(
        core_axis_name="c", subcore_axis_name="s",
        num_cores=NCORES, num_subcores=NT)
    out_type = jax.ShapeDtypeStruct((NCORES * N, 128), jnp.float32)
    scratch = [
        pltpu.VMEM((ES_EPT0,), jnp.int32),
        pltpu.VMEM((ES_EPT0 // CH, CH), jnp.int32),
        pltpu.VMEM((CH, 128), jnp.float32),
        pltpu.VMEM((ZR, 128), jnp.float32),
        pltpu.MemorySpace.VMEM_SHARED((N, 128), jnp.float32),
        pltpu.SemaphoreType.DMA,
    ]
    vpr = CH // 16
    SB = CH * 128 * 4

    def body(dst, out, dbuf, didx, ones, zbuf, acc, ssem):
        cid = lax.axis_index("c")
        sid = lax.axis_index("s")
        z16 = jnp.zeros((16,), jnp.float32)
        o16 = jnp.ones((16,), jnp.float32)

        def fill(i, _):
            r = i // 8
            c = i % 8
            zbuf[r, pl.ds(c * 16, 16)] = z16
            return 0
        lax.fori_loop(0, ZR * 8, fill, 0)

        def fo(i, _):
            r = i // 8
            c = i % 8
            ones[r, pl.ds(c * 16, 16)] = o16
            return 0
        lax.fori_loop(0, CH * 8, fo, 0)

        for z in range(NZ):
            pltpu.sync_copy(zbuf, acc.at[pl.ds(sid * FR + z * ZR, ZR)])
        pltpu.sync_copy(zbuf.at[pl.ds(0, ZREM)],
                        acc.at[pl.ds(sid * FR + NZ * ZR, ZREM)])

        @pl.when(sid == 0)
        def _():
            pltpu.sync_copy(zbuf.at[pl.ds(0, TAIL)],
                            acc.at[pl.ds(NT * FR, TAIL)])
        plsc.subcore_barrier()

        epc = ES_EPT0 - CH * cid
        base = cid * (NT * ES_EPT0) + sid * epc
        pltpu.sync_copy(dst.at[pl.ds(base, ES_EPT0 - CH)],
                        dbuf.at[pl.ds(0, ES_EPT0 - CH)])

        @pl.when(cid == 0)
        def _():
            pltpu.sync_copy(dst.at[pl.ds(base + ES_EPT0 - CH, CH)],
                            dbuf.at[pl.ds(ES_EPT0 - CH, CH)])

        def db(i, _):
            r = i // vpr
            c = (i % vpr) * 16
            didx[r, pl.ds(c, 16)] = dbuf[pl.ds(i * 16, 16)]
            return 0
        lax.fori_loop(0, ES_EPT0 // 16, db, 0)

        nst = epc // CH

        def fire(i, _):
            pltpu.async_copy(ones, acc.at[didx.at[i]], ssem, add=True)
            return 0
        lax.fori_loop(0, nst, fire, 0)

        def drain(i, _):
            pltpu.make_async_copy(ones, acc.at[didx.at[0]], ssem).wait()
            return 0
        lax.fori_loop(0, nst, drain, 0)
        plsc.subcore_barrier()
        pltpu.sync_copy(acc.at[pl.ds(sid * FR, FR)],
                        out.at[pl.ds(cid * N + sid * FR, FR)])

        @pl.when(sid == 0)
        def _():
            pltpu.sync_copy(acc.at[pl.ds(NT * FR, TAIL)],
                            out.at[pl.ds(cid * N + NT * FR, TAIL)])

    return pl.kernel(body, out_type=out_type, mesh=mesh,
                     scratch_types=scratch)


ES_EPT0 = 5040


def _sc_agg_esplit(dc):
    mesh = plsc.VectorSubcoreMesh(
        core_axis_name="c", subcore_axis_name="s",
        num_cores=NCORES, num_subcores=NT)
    out_type = jax.ShapeDtypeStruct((NCORES * N, dc), jnp.float32)
    scratch = [
        pltpu.VMEM((ES_EPT0,), jnp.int32),
        pltpu.VMEM((ES_EPT0,), jnp.int32),
        pltpu.VMEM((ES_EPT0 // CH, CH), jnp.int32),
        [pltpu.VMEM((CH, dc), jnp.float32) for _ in range(2)],
        pltpu.VMEM((ZR, dc), jnp.float32),
        pltpu.MemorySpace.VMEM_SHARED((N, dc), jnp.float32),
        [pltpu.SemaphoreType.DMA for _ in range(2)],
        pltpu.SemaphoreType.DMA,
    ]
    vpr = CH // 16

    def body(table, src, dst, out, sidx, dbuf, didx, rows, zbuf,
             acc, gsem, ssem):
        cid = lax.axis_index("c")
        sid = lax.axis_index("s")
        z16 = jnp.zeros((16,), jnp.float32)

        def zb(i, _):
            r = i // (dc // 16)
            c = i % (dc // 16)
            zbuf[r, pl.ds(c * 16, 16)] = z16
            return 0
        lax.fori_loop(0, ZR * (dc // 16), zb, 0)

        epc = ES_EPT0 - CH * cid
        base = cid * (NT * ES_EPT0) + sid * epc
        pltpu.sync_copy(src.at[pl.ds(base, ES_EPT0 - CH)],
                        sidx.at[pl.ds(0, ES_EPT0 - CH)])
        pltpu.sync_copy(dst.at[pl.ds(base, ES_EPT0 - CH)],
                        dbuf.at[pl.ds(0, ES_EPT0 - CH)])

        @pl.when(cid == 0)
        def _():
            pltpu.sync_copy(src.at[pl.ds(base + ES_EPT0 - CH, CH)],
                            sidx.at[pl.ds(ES_EPT0 - CH, CH)])
            pltpu.sync_copy(dst.at[pl.ds(base + ES_EPT0 - CH, CH)],
                            dbuf.at[pl.ds(ES_EPT0 - CH, CH)])

        def db(i, _):
            r = i // vpr
            c = (i % vpr) * 16
            didx[r, pl.ds(c, 16)] = dbuf[pl.ds(i * 16, 16)]
            return 0
        lax.fori_loop(0, ES_EPT0 // 16, db, 0)

        for z in range(NZ):
            pltpu.sync_copy(zbuf, acc.at[pl.ds(sid * FR + z * ZR, ZR)])
        pltpu.sync_copy(zbuf.at[pl.ds(0, ZREM)],
                        acc.at[pl.ds(sid * FR + NZ * ZR, ZREM)])

        @pl.when(sid == 0)
        def _():
            pltpu.sync_copy(zbuf.at[pl.ds(0, TAIL)],
                            acc.at[pl.ds(NT * FR, TAIL)])
        plsc.subcore_barrier()

        nst = epc // CH
        pltpu.async_copy(table.at[sidx.at[pl.ds(0, CH)]], rows[0], gsem[0])

        def pair(p, _):
            i0 = 2 * p
            pltpu.make_async_copy(
                table.at[sidx.at[pl.ds(0, CH)]], rows[0], gsem[0]).wait()
            pltpu.async_copy(
                table.at[sidx.at[pl.ds((i0 + 1) * CH, CH)]], rows[1],
                gsem[1])
            pltpu.async_copy(
                rows[0], acc.at[didx.at[i0]], ssem, add=True).wait()

            @pl.when(i0 + 2 < nst)
            def _():
                pltpu.async_copy(
                    table.at[sidx.at[pl.ds((i0 + 2) * CH, CH)]], rows[0],
                    gsem[0])
            pltpu.make_async_copy(
                table.at[sidx.at[pl.ds(0, CH)]], rows[1], gsem[1]).wait()
            pltpu.async_copy(
                rows[1], acc.at[didx.at[i0 + 1]], ssem, add=True).wait()
            return 0
        lax.fori_loop(0, (ES_EPT0 // CH - 1) // 2, pair, 0)

        @pl.when(cid == 0)
        def _():
            pltpu.make_async_copy(
                table.at[sidx.at[pl.ds(0, CH)]], rows[0], gsem[0]).wait()
            pltpu.async_copy(
                rows[0], acc.at[didx.at[ES_EPT0 // CH - 1]], ssem,
                add=True).wait()
        plsc.subcore_barrier()
        pltpu.sync_copy(acc.at[pl.ds(sid * FR, FR)],
                        out.at[pl.ds(cid * N + sid * FR, FR)])

        @pl.when(sid == 0)
        def _():
            pltpu.sync_copy(acc.at[pl.ds(NT * FR, TAIL)],
                            out.at[pl.ds(cid * N + NT * FR, TAIL)])

    return pl.kernel(body, out_type=out_type, mesh=mesh,
                     scratch_types=scratch)


def _tc_layer(nchi, dc, do, ncho, dco, relu, bn=400):
    di = nchi * dc

    def body(h_ref, s_ref, deg_ref, ws_ref, wn_ref, b_ref, out_ref):
        acc = jnp.zeros((bn, do), jnp.float32)
        accn = jnp.zeros((bn, do), jnp.float32)
        for c in range(nchi):
            acc = acc + jnp.dot(h_ref[c], ws_ref[pl.ds(c * dc, dc), :],
                                preferred_element_type=jnp.float32)
            accn = accn + jnp.dot(s_ref[c], wn_ref[pl.ds(c * dc, dc), :],
                                  preferred_element_type=jnp.float32)
        r = 1.0 / jnp.maximum(deg_ref[...], 1.0)
        res = acc + accn * r + b_ref[...]
        if relu:
            res = jnp.maximum(res, 0.0)
        for c2 in range(ncho):
            out_ref[c2] = res[:, c2 * dco:(c2 + 1) * dco]

    grid = (N // bn,)
    return pl.pallas_call(
        body,
        grid=grid,
        in_specs=[
            pl.BlockSpec((nchi, bn, dc), lambda i: (0, i, 0)),
            pl.BlockSpec((nchi, bn, dc), lambda i: (0, i, 0)),
            pl.BlockSpec((bn, 1), lambda i: (i, 0)),
            pl.BlockSpec((di, do), lambda i: (0, 0)),
            pl.BlockSpec((di, do), lambda i: (0, 0)),
            pl.BlockSpec((1, do), lambda i: (0, 0)),
        ],
        out_specs=pl.BlockSpec((ncho, bn, dco), lambda i: (0, i, 0)),
        out_shape=jax.ShapeDtypeStruct((ncho, N, dco), jnp.float32),
    )


def _tc_transform(nchi, dc, do, ncho, dco, bn=400):
    def body(h_ref, w_ref, out_ref):
        acc = jnp.zeros((bn, do), jnp.float32)
        for c in range(nchi):
            acc = acc + jnp.dot(h_ref[c], w_ref[pl.ds(c * dc, dc), :],
                                preferred_element_type=jnp.float32)
        for c2 in range(ncho):
            out_ref[c2] = acc[:, c2 * dco:(c2 + 1) * dco]

    return pl.pallas_call(
        body,
        grid=(N // bn,),
        in_specs=[
            pl.BlockSpec((nchi, bn, dc), lambda i: (0, i, 0)),
            pl.BlockSpec((nchi * dc, do), lambda i: (0, 0)),
        ],
        out_specs=pl.BlockSpec((ncho, bn, dco), lambda i: (0, i, 0)),
        out_shape=jax.ShapeDtypeStruct((ncho, N, dco), jnp.float32),
    )


def _tc_final(nchi, dc, do, bn=400):
    def body(h_ref, s_ref, deg_ref, ws_ref, b_ref, out_ref):
        acc = jnp.zeros((bn, do), jnp.float32)
        for c in range(nchi):
            acc = acc + jnp.dot(h_ref[c], ws_ref[pl.ds(c * dc, dc), :],
                                preferred_element_type=jnp.float32)
        mean = s_ref[0] + s_ref[1]
        r = 1.0 / jnp.maximum(deg_ref[...], 1.0)
        out_ref[...] = acc + mean * r + b_ref[...]

    return pl.pallas_call(
        body,
        grid=(N // bn,),
        in_specs=[
            pl.BlockSpec((nchi, bn, dc), lambda i: (0, i, 0)),
            pl.BlockSpec((2, bn, do), lambda i: (0, i, 0)),
            pl.BlockSpec((bn, 1), lambda i: (i, 0)),
            pl.BlockSpec((nchi * dc, do), lambda i: (0, 0)),
            pl.BlockSpec((1, do), lambda i: (0, 0)),
        ],
        out_specs=pl.BlockSpec((bn, do), lambda i: (i, 0)),
        out_shape=jax.ShapeDtypeStruct((N, do), jnp.float32),
    )


def kernel(x, edge_index, Ws0, Wn0, b0, Ws1, Wn1, b1, Ws2, Wn2, b2):
    src = edge_index[0]
    dst = edge_index[1]

    def srcf(nch):
        off = (jnp.arange(nch, dtype=jnp.int32) * N)[:, None]
        return (src[None, :] + off).reshape(nch * E)

    srcf2 = srcf(2)
    srcf4 = srcf(4)

    x_s = x.reshape(N, 2, 128).transpose(1, 0, 2)

    degf = _sc_deg()(dst)
    deg = degf[:N, :1] + degf[N:, :1]
    S0 = _sc_agg(2, 128)(x_s.reshape(2 * N, 128), srcf2, dst)

    h1_s = _tc_layer(2, 128, 512, 4, 128, True)(
        x_s, S0.reshape(2, N, 128), deg, Ws0, Wn0, b0.reshape(1, -1))

    S1 = _sc_agg(4, 128)(h1_s.reshape(4 * N, 128), srcf4, dst)

    h2_s = _tc_layer(4, 128, 512, 4, 128, True)(
        h1_s, S1.reshape(4, N, 128), deg, Ws1, Wn1, b1.reshape(1, -1))

    z2 = _tc_transform(4, 128, 128, 1, 128)(h2_s, Wn2)

    S2 = _sc_agg_esplit(128)(z2.reshape(N, 128), src, dst)

    out = _tc_final(4, 128, 128)(
        h2_s, S2.reshape(2, N, 128), deg, Ws2, b2.reshape(1, -1))
    return out


# speedup vs baseline: 8.1867x; 1.0331x over previous
import functools

import jax
import jax.numpy as jnp
from jax import lax
from jax.experimental import pallas as pl
from jax.experimental.pallas import tpu as pltpu
from jax.experimental.pallas import tpu_sc as plsc

N = 10000
E = 160000
NT = 16
NCORES = 2
CH = 80
EPT = E // NT
SCH = 2000
NSC = EPT // SCH
NKS = SCH // CH
FR = 624
TAIL = N - NT * FR
ZR = 80
NZ = FR // ZR
ZREM = FR - NZ * ZR


def _sc_agg(nch, dc):
    mesh = plsc.VectorSubcoreMesh(
        core_axis_name="c", subcore_axis_name="s",
        num_cores=NCORES, num_subcores=NT)
    out_type = jax.ShapeDtypeStruct((nch * N, dc), jnp.float32)
    scratch = [
        pltpu.VMEM((SCH,), jnp.int32),
        pltpu.VMEM((SCH,), jnp.int32),
        pltpu.VMEM((NKS, CH), jnp.int32),
        [pltpu.VMEM((CH, dc), jnp.float32) for _ in range(3)],
        pltpu.VMEM((ZR, dc), jnp.float32),
        pltpu.MemorySpace.VMEM_SHARED((N, dc), jnp.float32),
        [pltpu.SemaphoreType.DMA for _ in range(3)],
        [pltpu.SemaphoreType.DMA for _ in range(3)],
    ]
    vpr = CH // 16
    SB = CH * dc * 4

    def body(table, srcf, dst, out, sidx, dbuf, didx, rows, zbuf,
             acc, gsem, ssem):
        cid = lax.axis_index("c")
        sid = lax.axis_index("s")
        z16 = jnp.zeros((16,), jnp.float32)

        def zb(i, _):
            r = i // (dc // 16)
            c = i % (dc // 16)
            zbuf[r, pl.ds(c * 16, 16)] = z16
            return 0
        lax.fori_loop(0, ZR * (dc // 16), zb, 0)

        for j in range(nch // NCORES):
            cch = NCORES * j + cid
            for z in range(NZ):
                pltpu.sync_copy(zbuf, acc.at[pl.ds(sid * FR + z * ZR, ZR)])
            pltpu.sync_copy(zbuf.at[pl.ds(0, ZREM)],
                            acc.at[pl.ds(sid * FR + NZ * ZR, ZREM)])

            @pl.when(sid == 0)
            def _():
                pltpu.sync_copy(zbuf.at[pl.ds(0, TAIL)],
                                acc.at[pl.ds(NT * FR, TAIL)])
            plsc.subcore_barrier()

            def sc_body(s, _):
                ebase = sid * EPT + s * SCH
                pltpu.sync_copy(srcf.at[pl.ds(cch * E + ebase, SCH)], sidx)
                pltpu.async_copy(
                    table.at[sidx.at[pl.ds(0, CH)]], rows[0], gsem[0])
                pltpu.sync_copy(dst.at[pl.ds(ebase, SCH)], dbuf)

                def db(i, _):
                    r = i // vpr
                    c = (i % vpr) * 16
                    didx[r, pl.ds(c, 16)] = dbuf[pl.ds(i * 16, 16)]
                    return 0
                lax.fori_loop(0, SCH // 16, db, 0)

                pltpu.async_copy(
                    table.at[sidx.at[pl.ds(CH, CH)]], rows[1], gsem[1])
                pltpu.async_copy(zbuf, acc.at[didx.at[0]], ssem[2],
                                 add=True)

                def gwait(b):
                    pltpu.make_async_copy(
                        table.at[sidx.at[pl.ds(0, CH)]], rows[b],
                        gsem[b]).wait()

                def swait(b):
                    pltpu.make_async_copy(
                        rows[b], acc.at[didx.at[0]], ssem[b]).wait()

                def tri(q, _):
                    for k in range(3):
                        i = 3 * q + k
                        bp = (k - 1) % 3
                        gwait(k)
                        pltpu.async_copy(rows[k], acc.at[didx.at[i]],
                                         ssem[k], add=True)
                        swait(bp)

                        @pl.when(i + 2 < NKS)
                        def _():
                            pltpu.async_copy(
                                table.at[sidx.at[pl.ds((i + 2) * CH, CH)]],
                                rows[bp], gsem[bp])
                    return 0
                lax.fori_loop(0, NKS // 3, tri, 0)
                gwait(0)
                pltpu.async_copy(rows[0], acc.at[didx.at[NKS - 1]],
                                 ssem[0], add=True)
                swait(2)
                swait(0)
                return 0
            lax.fori_loop(0, NSC, sc_body, 0)
            plsc.subcore_barrier()
            pltpu.sync_copy(
                acc.at[pl.ds(sid * FR, FR)],
                out.at[pl.ds(cch * N + sid * FR, FR)])

            @pl.when(sid == 0)
            def _():
                pltpu.sync_copy(acc.at[pl.ds(NT * FR, TAIL)],
                                out.at[pl.ds(cch * N + NT * FR, TAIL)])
            plsc.subcore_barrier()

    return pl.kernel(body, out_type=out_type, mesh=mesh,
                     scratch_types=scratch)


def _sc_deg():
    mesh = plsc.VectorSubcoreMesh(
        core_axis_name="c", subcore_axis_name="s",
        num_cores=NCORES, num_subcores=NT)
    out_type = jax.ShapeDtypeStruct((NCORES * N, 128), jnp.float32)
    scratch = [
        pltpu.VMEM((ES_EPT0,), jnp.int32),
        pltpu.VMEM((ES_EPT0 // CH, CH), jnp.int32),
        pltpu.VMEM((CH, 128), jnp.float32),
        pltpu.VMEM((ZR, 128), jnp.float32),
        pltpu.MemorySpace.VMEM_SHARED((N, 128), jnp.float32),
        pltpu.SemaphoreType.DMA,
    ]
    vpr = CH // 16
    SB = CH * 128 * 4

    def body(dst, out, dbuf, didx, ones, zbuf, acc, ssem):
        cid = lax.axis_index("c")
        sid = lax.axis_index("s")
        z16 = jnp.zeros((16,), jnp.float32)
        o16 = jnp.ones((16,), jnp.float32)

        def fill(i, _):
            r = i // 8
            c = i % 8
            zbuf[r, pl.ds(c * 16, 16)] = z16
            return 0
        lax.fori_loop(0, ZR * 8, fill, 0)

        def fo(i, _):
            r = i // 8
            c = i % 8
            ones[r, pl.ds(c * 16, 16)] = o16
            return 0
        lax.fori_loop(0, CH * 8, fo, 0)

        for z in range(NZ):
            pltpu.sync_copy(zbuf, acc.at[pl.ds(sid * FR + z * ZR, ZR)])
        pltpu.sync_copy(zbuf.at[pl.ds(0, ZREM)],
                        acc.at[pl.ds(sid * FR + NZ * ZR, ZREM)])

        @pl.when(sid == 0)
        def _():
            pltpu.sync_copy(zbuf.at[pl.ds(0, TAIL)],
                            acc.at[pl.ds(NT * FR, TAIL)])
        plsc.subcore_barrier()

        epc = ES_EPT0 - CH * cid
        base = cid * (NT * ES_EPT0) + sid * epc
        pltpu.sync_copy(dst.at[pl.ds(base, ES_EPT0 - CH)],
                        dbuf.at[pl.ds(0, ES_EPT0 - CH)])

        @pl.when(cid == 0)
        def _():
            pltpu.sync_copy(dst.at[pl.ds(base + ES_EPT0 - CH, CH)],
                            dbuf.at[pl.ds(ES_EPT0 - CH, CH)])

        def db(i, _):
            r = i // vpr
            c = (i % vpr) * 16
            didx[r, pl.ds(c, 16)] = dbuf[pl.ds(i * 16, 16)]
            return 0
        lax.fori_loop(0, ES_EPT0 // 16, db, 0)

        nst = epc // CH

        def fire(i, _):
            pltpu.async_copy(ones, acc.at[didx.at[i]], ssem, add=True)
            return 0
        lax.fori_loop(0, nst, fire, 0)

        def drain(i, _):
            pltpu.make_async_copy(ones, acc.at[didx.at[0]], ssem).wait()
            return 0
        lax.fori_loop(0, nst, drain, 0)
        plsc.subcore_barrier()
        pltpu.sync_copy(acc.at[pl.ds(sid * FR, FR)],
                        out.at[pl.ds(cid * N + sid * FR, FR)])

        @pl.when(sid == 0)
        def _():
            pltpu.sync_copy(acc.at[pl.ds(NT * FR, TAIL)],
                            out.at[pl.ds(cid * N + NT * FR, TAIL)])

    return pl.kernel(body, out_type=out_type, mesh=mesh,
                     scratch_types=scratch)


ES_EPT0 = 5040


def _sc_agg_esplit(dc):
    mesh = plsc.VectorSubcoreMesh(
        core_axis_name="c", subcore_axis_name="s",
        num_cores=NCORES, num_subcores=NT)
    out_type = jax.ShapeDtypeStruct((NCORES * N, dc), jnp.float32)
    scratch = [
        pltpu.VMEM((ES_EPT0,), jnp.int32),
        pltpu.VMEM((ES_EPT0,), jnp.int32),
        pltpu.VMEM((ES_EPT0 // CH, CH), jnp.int32),
        [pltpu.VMEM((CH, dc), jnp.float32) for _ in range(2)],
        pltpu.VMEM((ZR, dc), jnp.float32),
        pltpu.MemorySpace.VMEM_SHARED((N, dc), jnp.float32),
        [pltpu.SemaphoreType.DMA for _ in range(2)],
        pltpu.SemaphoreType.DMA,
    ]
    vpr = CH // 16

    def body(table, src, dst, out, sidx, dbuf, didx, rows, zbuf,
             acc, gsem, ssem):
        cid = lax.axis_index("c")
        sid = lax.axis_index("s")
        z16 = jnp.zeros((16,), jnp.float32)

        def zb(i, _):
            r = i // (dc // 16)
            c = i % (dc // 16)
            zbuf[r, pl.ds(c * 16, 16)] = z16
            return 0
        lax.fori_loop(0, ZR * (dc // 16), zb, 0)

        epc = ES_EPT0 - CH * cid
        base = cid * (NT * ES_EPT0) + sid * epc
        pltpu.sync_copy(src.at[pl.ds(base, ES_EPT0 - CH)],
                        sidx.at[pl.ds(0, ES_EPT0 - CH)])
        pltpu.sync_copy(dst.at[pl.ds(base, ES_EPT0 - CH)],
                        dbuf.at[pl.ds(0, ES_EPT0 - CH)])

        @pl.when(cid == 0)
        def _():
            pltpu.sync_copy(src.at[pl.ds(base + ES_EPT0 - CH, CH)],
                            sidx.at[pl.ds(ES_EPT0 - CH, CH)])
            pltpu.sync_copy(dst.at[pl.ds(base + ES_EPT0 - CH, CH)],
                            dbuf.at[pl.ds(ES_EPT0 - CH, CH)])

        def db(i, _):
            r = i // vpr
            c = (i % vpr) * 16
            didx[r, pl.ds(c, 16)] = dbuf[pl.ds(i * 16, 16)]
            return 0
        lax.fori_loop(0, ES_EPT0 // 16, db, 0)

        for z in range(NZ):
            pltpu.sync_copy(zbuf, acc.at[pl.ds(sid * FR + z * ZR, ZR)])
        pltpu.sync_copy(zbuf.at[pl.ds(0, ZREM)],
                        acc.at[pl.ds(sid * FR + NZ * ZR, ZREM)])

        @pl.when(sid == 0)
        def _():
            pltpu.sync_copy(zbuf.at[pl.ds(0, TAIL)],
                            acc.at[pl.ds(NT * FR, TAIL)])
        plsc.subcore_barrier()

        nst = epc // CH
        pltpu.async_copy(table.at[sidx.at[pl.ds(0, CH)]], rows[0], gsem[0])

        def pair(p, _):
            i0 = 2 * p
            pltpu.make_async_copy(
                table.at[sidx.at[pl.ds(0, CH)]], rows[0], gsem[0]).wait()
            pltpu.async_copy(
                table.at[sidx.at[pl.ds((i0 + 1) * CH, CH)]], rows[1],
                gsem[1])
            pltpu.async_copy(
                rows[0], acc.at[didx.at[i0]], ssem, add=True).wait()

            @pl.when(i0 + 2 < nst)
            def _():
                pltpu.async_copy(
                    table.at[sidx.at[pl.ds((i0 + 2) * CH, CH)]], rows[0],
                    gsem[0])
            pltpu.make_async_copy(
                table.at[sidx.at[pl.ds(0, CH)]], rows[1], gsem[1]).wait()
            pltpu.async_copy(
                rows[1], acc.at[didx.at[i0 + 1]], ssem, add=True).wait()
            return 0
        lax.fori_loop(0, (ES_EPT0 // CH - 1) // 2, pair, 0)

        @pl.when(cid == 0)
        def _():
            pltpu.make_async_copy(
                table.at[sidx.at[pl.ds(0, CH)]], rows[0], gsem[0]).wait()
            pltpu.async_copy(
                rows[0], acc.at[didx.at[ES_EPT0 // CH - 1]], ssem,
                add=True).wait()
        plsc.subcore_barrier()
        pltpu.sync_copy(acc.at[pl.ds(sid * FR, FR)],
                        out.at[pl.ds(cid * N + sid * FR, FR)])

        @pl.when(sid == 0)
        def _():
            pltpu.sync_copy(acc.at[pl.ds(NT * FR, TAIL)],
                            out.at[pl.ds(cid * N + NT * FR, TAIL)])

    return pl.kernel(body, out_type=out_type, mesh=mesh,
                     scratch_types=scratch)


def _tc_layer(nchi, dc, do, ncho, dco, relu, bn=400):
    di = nchi * dc

    def body(h_ref, s_ref, deg_ref, ws_ref, wn_ref, b_ref, out_ref):
        acc = jnp.zeros((bn, do), jnp.float32)
        accn = jnp.zeros((bn, do), jnp.float32)
        for c in range(nchi):
            acc = acc + jnp.dot(h_ref[c], ws_ref[pl.ds(c * dc, dc), :],
                                preferred_element_type=jnp.float32)
            accn = accn + jnp.dot(s_ref[c], wn_ref[pl.ds(c * dc, dc), :],
                                  preferred_element_type=jnp.float32)
        r = 1.0 / jnp.maximum(deg_ref[...], 1.0)
        res = acc + accn * r + b_ref[...]
        if relu:
            res = jnp.maximum(res, 0.0)
        for c2 in range(ncho):
            out_ref[c2] = res[:, c2 * dco:(c2 + 1) * dco]

    grid = (N // bn,)
    return pl.pallas_call(
        body,
        grid=grid,
        in_specs=[
            pl.BlockSpec((nchi, bn, dc), lambda i: (0, i, 0)),
            pl.BlockSpec((nchi, bn, dc), lambda i: (0, i, 0)),
            pl.BlockSpec((bn, 1), lambda i: (i, 0)),
            pl.BlockSpec((di, do), lambda i: (0, 0)),
            pl.BlockSpec((di, do), lambda i: (0, 0)),
            pl.BlockSpec((1, do), lambda i: (0, 0)),
        ],
        out_specs=pl.BlockSpec((ncho, bn, dco), lambda i: (0, i, 0)),
        out_shape=jax.ShapeDtypeStruct((ncho, N, dco), jnp.float32),
    )


def _tc_layer_fused(nchi, dc, do, ncho, dco, do2, bn=400):
    di = nchi * dc

    def body(h_ref, s_ref, deg_ref, ws_ref, wn_ref, b_ref, w2_ref,
             out_ref, z_ref):
        acc = jnp.zeros((bn, do), jnp.float32)
        accn = jnp.zeros((bn, do), jnp.float32)
        for c in range(nchi):
            acc = acc + jnp.dot(h_ref[c], ws_ref[pl.ds(c * dc, dc), :],
                                preferred_element_type=jnp.float32)
            accn = accn + jnp.dot(s_ref[c], wn_ref[pl.ds(c * dc, dc), :],
                                  preferred_element_type=jnp.float32)
        r = 1.0 / jnp.maximum(deg_ref[...], 1.0)
        res = jnp.maximum(acc + accn * r + b_ref[...], 0.0)
        for c2 in range(ncho):
            out_ref[c2] = res[:, c2 * dco:(c2 + 1) * dco]
        z_ref[...] = jnp.dot(res, w2_ref[...],
                             preferred_element_type=jnp.float32)

    return pl.pallas_call(
        body,
        grid=(N // bn,),
        in_specs=[
            pl.BlockSpec((nchi, bn, dc), lambda i: (0, i, 0)),
            pl.BlockSpec((nchi, bn, dc), lambda i: (0, i, 0)),
            pl.BlockSpec((bn, 1), lambda i: (i, 0)),
            pl.BlockSpec((di, do), lambda i: (0, 0)),
            pl.BlockSpec((di, do), lambda i: (0, 0)),
            pl.BlockSpec((1, do), lambda i: (0, 0)),
            pl.BlockSpec((do, do2), lambda i: (0, 0)),
        ],
        out_specs=[
            pl.BlockSpec((ncho, bn, dco), lambda i: (0, i, 0)),
            pl.BlockSpec((bn, do2), lambda i: (i, 0)),
        ],
        out_shape=[
            jax.ShapeDtypeStruct((ncho, N, dco), jnp.float32),
            jax.ShapeDtypeStruct((N, do2), jnp.float32),
        ],
    )


def _tc_final(nchi, dc, do, bn=400):
    def body(h_ref, s_ref, deg_ref, ws_ref, b_ref, out_ref):
        acc = jnp.zeros((bn, do), jnp.float32)
        for c in range(nchi):
            acc = acc + jnp.dot(h_ref[c], ws_ref[pl.ds(c * dc, dc), :],
                                preferred_element_type=jnp.float32)
        mean = s_ref[0] + s_ref[1]
        r = 1.0 / jnp.maximum(deg_ref[...], 1.0)
        out_ref[...] = acc + mean * r + b_ref[...]

    return pl.pallas_call(
        body,
        grid=(N // bn,),
        in_specs=[
            pl.BlockSpec((nchi, bn, dc), lambda i: (0, i, 0)),
            pl.BlockSpec((2, bn, do), lambda i: (0, i, 0)),
            pl.BlockSpec((bn, 1), lambda i: (i, 0)),
            pl.BlockSpec((nchi * dc, do), lambda i: (0, 0)),
            pl.BlockSpec((1, do), lambda i: (0, 0)),
        ],
        out_specs=pl.BlockSpec((bn, do), lambda i: (i, 0)),
        out_shape=jax.ShapeDtypeStruct((N, do), jnp.float32),
    )


def kernel(x, edge_index, Ws0, Wn0, b0, Ws1, Wn1, b1, Ws2, Wn2, b2):
    src = edge_index[0]
    dst = edge_index[1]

    def srcf(nch):
        off = (jnp.arange(nch, dtype=jnp.int32) * N)[:, None]
        return (src[None, :] + off).reshape(nch * E)

    srcf2 = srcf(2)
    srcf4 = srcf(4)

    x_s = x.reshape(N, 2, 128).transpose(1, 0, 2)

    degf = _sc_deg()(dst)
    deg = degf[:N, :1] + degf[N:, :1]
    S0 = _sc_agg(2, 128)(x_s.reshape(2 * N, 128), srcf2, dst)

    h1_s = _tc_layer(2, 128, 512, 4, 128, True)(
        x_s, S0.reshape(2, N, 128), deg, Ws0, Wn0, b0.reshape(1, -1))

    S1 = _sc_agg(4, 128)(h1_s.reshape(4 * N, 128), srcf4, dst)

    h2_s, z2 = _tc_layer_fused(4, 128, 512, 4, 128, 128)(
        h1_s, S1.reshape(4, N, 128), deg, Ws1, Wn1, b1.reshape(1, -1), Wn2)

    S2 = _sc_agg_esplit(128)(z2, src, dst)

    out = _tc_final(4, 128, 128)(
        h2_s, S2.reshape(2, N, 128), deg, Ws2, b2.reshape(1, -1))
    return out
